# Initial kernel scaffold; baseline (speedup 1.0000x reference)
import jax
import jax.numpy as jnp
from jax.experimental import pallas as pl


def kernel(xe_patch, ye_patch, spatial):
    raise NotImplementedError("write your pallas kernel here")


# speedup vs baseline: 5.8134x; 5.8134x over previous
import functools

import jax
import jax.numpy as jnp
from jax import lax
from jax.experimental import pallas as pl
from jax.experimental.pallas import tpu as pltpu
from jax.experimental.pallas import tpu_sc as plsc

_N = 4096
_E = 256
_K = 16
_SCALE = 2
_BN = 256
_NW = 32
_EW = _E // _NW


def _topk_body(xe_ref, ye_ref, score_ref, idx_ref, d2_ref):
    xb = xe_ref[...]
    y = ye_ref[...]
    x2 = jnp.sum(xb * xb, axis=1, keepdims=True)
    y2 = jnp.sum(y * y, axis=1)
    prod = lax.dot_general(xb, y, (((1,), (1,)), ((), ())),
                           preferred_element_type=jnp.float32)
    d2 = x2 + y2[None, :] - 2.0 * prod
    d2_ref[...] = jnp.maximum(d2, jnp.float32(1e-12))
    iota = lax.broadcasted_iota(jnp.int32, (_BN, _N), 1)
    scores = []
    idxs = []
    for _ in range(_K):
        d = d2_ref[...]
        mv = jnp.min(d, axis=1)
        eq = d == mv[:, None]
        mi = jnp.min(jnp.where(eq, iota, jnp.int32(_N)), axis=1)
        scores.append(jnp.exp(-jnp.sqrt(mv) / 10.0))
        idxs.append(mi)
        d2_ref[...] = jnp.where(iota == mi[:, None], jnp.float32(jnp.inf), d)
    score_ref[...] = jnp.stack(scores, axis=1)
    idx_ref[...] = jnp.stack(idxs, axis=1)


def _topk(xe, ye):
    return pl.pallas_call(
        _topk_body,
        grid=(_N // _BN,),
        in_specs=[
            pl.BlockSpec((_BN, _E), lambda i: (i, 0)),
            pl.BlockSpec((_N, _E), lambda i: (0, 0)),
        ],
        out_specs=[
            pl.BlockSpec((_BN, _K), lambda i: (i, 0)),
            pl.BlockSpec((_BN, _K), lambda i: (i, 0)),
        ],
        out_shape=[
            jax.ShapeDtypeStruct((_N, _K), jnp.float32),
            jax.ShapeDtypeStruct((_N, _K), jnp.int32),
        ],
        scratch_shapes=[pltpu.VMEM((_BN, _N), jnp.float32)],
    )(xe, ye)


def _sc_diff_body(xet_hbm, yet_hbm, idxt_hbm, out_hbm,
                  xcols, ycols, idxv, outbuf):
    wid = lax.axis_index("s") * 2 + lax.axis_index("c")
    ee0 = wid * _EW
    pltpu.sync_copy(xet_hbm.at[pl.ds(ee0, _EW)], xcols)
    pltpu.sync_copy(yet_hbm.at[pl.ds(ee0, _EW)], ycols)

    def kk_body(kk, carry):
        pltpu.sync_copy(idxt_hbm.at[kk], idxv)

        def i_body(iv, c):
            sl = pl.ds(iv * 16, 16)
            idx16 = idxv[sl]
            for ee in range(_EW):
                row = jnp.full((16,), ee, jnp.int32)
                g = plsc.load_gather(xcols, [row, idx16])
                outbuf[ee, sl] = jnp.abs(ycols[ee, sl] - g)
            return c

        lax.fori_loop(0, _N // 16, i_body, 0, unroll=2)
        base = kk * _E + ee0
        pltpu.sync_copy(outbuf, out_hbm.at[pl.ds(base, _EW), pl.ds(0, _N)])
        pltpu.sync_copy(outbuf, out_hbm.at[pl.ds(base, _EW), pl.ds(_N, _N)])
        return carry

    lax.fori_loop(0, _K, kk_body, 0)


@functools.partial(jax.jit, static_argnames=())
def _sc_diff(xet, yet, idxt):
    mesh = plsc.VectorSubcoreMesh(core_axis_name="c", subcore_axis_name="s",
                                  num_cores=2, num_subcores=16)
    return pl.kernel(
        _sc_diff_body,
        out_type=jax.ShapeDtypeStruct((_K * _E, _SCALE * _N), jnp.float32),
        mesh=mesh,
        scratch_types=[
            pltpu.VMEM((_EW, _N), jnp.float32),
            pltpu.VMEM((_EW, _N), jnp.float32),
            pltpu.VMEM((_N,), jnp.int32),
            pltpu.VMEM((_EW, _N), jnp.float32),
        ],
        compiler_params=pltpu.CompilerParams(use_tc_tiling_on_sc=False,
                                             needs_layout_passes=False),
    )(xet, yet, idxt)


def kernel(xe_patch, ye_patch, spatial):
    del spatial
    score, idx = _topk(xe_patch, ye_patch)
    diff = _sc_diff(xe_patch.T, ye_patch.T, idx.T)
    sk = jnp.broadcast_to(score.T[None, :, None, :],
                          (1, _K, _SCALE, _N)).reshape(1, _K, _SCALE * _N)
    return sk, idx[None], diff[None]


# speedup vs baseline: 6.1690x; 1.0612x over previous
import jax
import jax.numpy as jnp
from jax import lax
from jax.experimental import pallas as pl
from jax.experimental.pallas import tpu as pltpu
from jax.experimental.pallas import tpu_sc as plsc

_N = 4096
_E = 256
_K = 16
_SCALE = 2
_BN = 256
_NW = 32
_EW = 4


def _topk_body(xe_ref, ye_ref, score_ref, idx_ref, idxt_ref, xet_ref,
               yet_ref, d2_ref):
    i = pl.program_id(0)
    xb = xe_ref[...]
    y = ye_ref[...]
    x2 = jnp.sum(xb * xb, axis=1, keepdims=True)
    y2 = jnp.sum(y * y, axis=1)
    prod = lax.dot_general(xb, y, (((1,), (1,)), ((), ())),
                           preferred_element_type=jnp.float32)
    d2 = x2 + y2[None, :] - 2.0 * prod
    d2_ref[...] = jnp.maximum(d2, jnp.float32(1e-12))
    xet_ref[...] = xb.T
    yet_ref[...] = ye_ref[pl.ds(i * _BN, _BN), :].T
    iota_f = lax.broadcasted_iota(jnp.int32, (_BN, _N), 1).astype(jnp.float32)
    scores = []
    idxs = []
    for _ in range(_K):
        d = d2_ref[...]
        mv = jnp.min(d, axis=1)
        t = jnp.where(d == mv[:, None], iota_f, jnp.float32(_N))
        mif = jnp.min(t, axis=1)
        d2_ref[...] = jnp.where(t == mif[:, None], jnp.float32(jnp.inf), d)
        scores.append(jnp.exp(-jnp.sqrt(mv) / 10.0))
        idxs.append(mif.astype(jnp.int32))
    score_ref[...] = jnp.stack(scores, axis=0)
    idx_ref[...] = jnp.stack(idxs, axis=1)
    idxt_ref[...] = jnp.stack(idxs, axis=0)


def _topk(xe, ye):
    return pl.pallas_call(
        _topk_body,
        grid=(_N // _BN,),
        in_specs=[
            pl.BlockSpec((_BN, _E), lambda i: (i, 0)),
            pl.BlockSpec((_N, _E), lambda i: (0, 0)),
        ],
        out_specs=[
            pl.BlockSpec((_K, _BN), lambda i: (0, i)),
            pl.BlockSpec((_BN, _K), lambda i: (i, 0)),
            pl.BlockSpec((_K, _BN), lambda i: (0, i)),
            pl.BlockSpec((_E, _BN), lambda i: (0, i)),
            pl.BlockSpec((_E, _BN), lambda i: (0, i)),
        ],
        out_shape=[
            jax.ShapeDtypeStruct((_K, _N), jnp.float32),
            jax.ShapeDtypeStruct((_N, _K), jnp.int32),
            jax.ShapeDtypeStruct((_K, _N), jnp.int32),
            jax.ShapeDtypeStruct((_E, _N), jnp.float32),
            jax.ShapeDtypeStruct((_E, _N), jnp.float32),
        ],
        scratch_shapes=[pltpu.VMEM((_BN, _N), jnp.float32)],
    )(xe, ye)


def _sc_diff_body(xet_hbm, yet_hbm, idxt_hbm, out_hbm,
                  xcols, ycols, idxv, ob0, ob1, sem0, sem1):
    wid = lax.axis_index("s") * 2 + lax.axis_index("c")

    for h in range(2):
        ee0 = wid * (2 * _EW) + h * _EW
        pltpu.sync_copy(xet_hbm.at[pl.ds(ee0, _EW)], xcols)
        pltpu.sync_copy(yet_hbm.at[pl.ds(ee0, _EW)], ycols)

        def kk2_body(kk2, carry):
            for ob, sem, b in ((ob0, sem0, 0), (ob1, sem1, 1)):
                kk = kk2 * 2 + b
                base = kk * _E + ee0
                dst0 = out_hbm.at[pl.ds(base, _EW), pl.ds(0, _N)]
                dst1 = out_hbm.at[pl.ds(base, _EW), pl.ds(_N, _N)]

                @pl.when(kk2 > 0)
                def _drain():
                    pltpu.make_async_copy(ob, dst0, sem).wait()
                    pltpu.make_async_copy(ob, dst1, sem).wait()

                pltpu.sync_copy(idxt_hbm.at[kk], idxv)

                def i_body(iv, c):
                    sl = pl.ds(iv * 16, 16)
                    idx16 = idxv[sl]
                    for ee in range(_EW):
                        row = jnp.full((16,), ee, jnp.int32)
                        g = plsc.load_gather(xcols, [row, idx16])
                        ob[ee, sl] = jnp.abs(ycols[ee, sl] - g)
                    return c

                lax.fori_loop(0, _N // 16, i_body, 0, unroll=4)
                pltpu.make_async_copy(ob, dst0, sem).start()
                pltpu.make_async_copy(ob, dst1, sem).start()
            return carry

        lax.fori_loop(0, _K // 2, kk2_body, 0)
        for ob, sem, b in ((ob0, sem0, 0), (ob1, sem1, 1)):
            kk = _K - 2 + b
            base = kk * _E + ee0
            pltpu.make_async_copy(
                ob, out_hbm.at[pl.ds(base, _EW), pl.ds(0, _N)], sem).wait()
            pltpu.make_async_copy(
                ob, out_hbm.at[pl.ds(base, _EW), pl.ds(_N, _N)], sem).wait()


def _sc_diff(xet, yet, idxt):
    mesh = plsc.VectorSubcoreMesh(core_axis_name="c", subcore_axis_name="s",
                                  num_cores=2, num_subcores=16)
    return pl.kernel(
        _sc_diff_body,
        out_type=jax.ShapeDtypeStruct((_K * _E, _SCALE * _N), jnp.float32),
        mesh=mesh,
        scratch_types=[
            pltpu.VMEM((_EW, _N), jnp.float32),
            pltpu.VMEM((_EW, _N), jnp.float32),
            pltpu.VMEM((_N,), jnp.int32),
            pltpu.VMEM((_EW, _N), jnp.float32),
            pltpu.VMEM((_EW, _N), jnp.float32),
            pltpu.SemaphoreType.DMA,
            pltpu.SemaphoreType.DMA,
        ],
        compiler_params=pltpu.CompilerParams(use_tc_tiling_on_sc=False,
                                             needs_layout_passes=False),
    )(xet, yet, idxt)


def kernel(xe_patch, ye_patch, spatial):
    del spatial
    score_t, idx, idxt, xet, yet = _topk(xe_patch, ye_patch)
    diff = _sc_diff(xet, yet, idxt)
    sk = jnp.concatenate([score_t, score_t], axis=1)[None]
    return sk, idx[None], diff[None]
